# baseline (speedup 1.0000x reference)
import functools

import jax
import jax.numpy as jnp
from jax import lax
from jax.experimental import pallas as pl
from jax.experimental.pallas import tpu as pltpu
from jax.experimental.pallas import tpu_sc as plsc

NC = 2
NS = 16
L = 16
NW = NC * NS

NUM_IMGS = 16
SPATIAL = 512 * 512
GROUPS = ((0, 4), (4, 4), (8, 4), (12, 3), (15, 1))
U = 8
TBL = 17 * L
TBL_P = 24 * L


def _tc_stage_body(p_ref, t_ref, x_ref, e_ref):
    p = p_ref[0]
    t = t_ref[0]
    x = x_ref[0]
    d = (0.299 * (p[0] - t[0]) + 0.587 * (p[1] - t[1])
         + 0.114 * (p[2] - t[2]))
    l16 = 4.784 * x[0] + 9.392 * x[1] + 1.824 * x[2]
    b = jnp.minimum(l16.astype(jnp.int32), 16)
    enc = (lax.bitcast_convert_type(d, jnp.int32) & ~31) | b
    for ct in range(4):
        sl = slice(128 * ct, 128 * (ct + 1))
        e_ref[0, :, ct] = enc[:, sl].reshape(64, 8, 128)


def _make_sc_body(pix_per_tile, ch, steps, nbuf):
    def _sc_body(e_hbm, out_ref, *scratch):
        wid = lax.axis_index("s") * NC + lax.axis_index("c")
        base = pl.multiple_of(wid * pix_per_tile, 8)
        ebufs = scratch[:nbuf]
        accs = scratch[nbuf:nbuf + 4]
        stage = scratch[nbuf + 4]
        sems = scratch[nbuf + 5:]

        zero = jnp.zeros((L,), jnp.float32)
        for acc in accs:
            for j in range(TBL_P // L):
                acc[pl.ds(j * L, L)] = zero

        lane = lax.iota(jnp.int32, L)
        ones = jnp.ones((L,), jnp.float32)

        def start_dmas(step, slot):
            off = pl.multiple_of(base + step * ch, 8)
            pltpu.make_async_copy(e_hbm.at[pl.ds(off, ch)], ebufs[slot],
                                  sems[slot]).start()

        def wait_dmas(slot):
            pltpu.make_async_copy(e_hbm.at[pl.ds(0, ch)], ebufs[slot],
                                  sems[slot]).wait()

        def compute(slot):
            ebuf = ebufs[slot]

            @plsc.parallel_loop(0, ch // L, step=2, unroll=U // 2)
            def px(j):
                for u in range(2):
                    ev = ebuf[pl.ds((j + u) * L, L)]
                    idx = (ev & 31) * L + lane
                    dv = plsc.bitcast(ev & ~31, jnp.float32)
                    plsc.addupdate_scatter(accs[2 * u], [idx], ones)
                    plsc.addupdate_scatter(accs[2 * u + 1], [idx], dv)

        for s in range(nbuf - 1):
            start_dmas(s, s)

        def step_body(t, _):
            step0 = t * nbuf
            for slot in range(nbuf):
                step = step0 + slot
                nxt = step + nbuf - 1

                @pl.when(nxt < steps)
                def _start_next():
                    start_dmas(nxt, (nbuf - 1 + slot) % nbuf)

                wait_dmas(slot)
                compute(slot)
            return _

        lax.fori_loop(0, steps // nbuf, step_body, None)

        for q, acc in enumerate(accs):
            for j in range(TBL_P // L):
                stage[q * 3 + j // 8, pl.ds((j % 8) * L, L)] = (
                    acc[pl.ds(j * L, L)])
        pltpu.sync_copy(stage, out_ref.at[pl.ds(wid * 16, 16)])

    return _sc_body


def _finish_body(*refs):
    o_ref = refs[-1]
    x = refs[0][...]
    for r in refs[1:-1]:
        x = x + r[...]
    s = jnp.sum(x.reshape(NW, 16, 128), axis=0)
    cnt = s[0:3] + s[6:9]
    dd = s[3:6] + s[9:12]
    loss = jnp.zeros((1, 1), jnp.float32)
    for b in range(16):
        r = b // 8
        c0 = (b % 8) * L
        cb = jnp.sum(cnt[r:r + 1, c0:c0 + L], axis=1, keepdims=True)
        db = jnp.sum(dd[r:r + 1, c0:c0 + L], axis=1, keepdims=True)
        loss = loss + jnp.where(cb > 0.0,
                                jnp.abs(db) / jnp.maximum(cb, 1.0),
                                jnp.zeros_like(cb))
    o_ref[...] = loss / 16.0


def kernel(pred, target, input_img):
    f32 = jnp.float32
    mesh = plsc.VectorSubcoreMesh(core_axis_name="c", subcore_axis_name="s",
                                  num_cores=NC, num_subcores=NS)
    sc_hists = {}
    for _, ipg in GROUPS:
        if ipg in sc_hists:
            continue
        ppt = ipg * SPATIAL // NW
        ch = 16384 if ppt % 16384 == 0 else 8192
        steps = ppt // ch
        nbuf = 2 if steps % 2 == 0 else steps
        sc_hists[ipg] = pl.kernel(
            _make_sc_body(ppt, ch, steps, nbuf),
            out_type=jax.ShapeDtypeStruct((NW * 16, 128), f32),
            mesh=mesh,
            compiler_params=pltpu.CompilerParams(needs_layout_passes=False),
            scratch_types=(
                [pltpu.VMEM((ch,), jnp.int32)] * nbuf
                + [pltpu.VMEM((TBL_P,), f32)] * 4
                + [pltpu.VMEM((16, 128), f32)]
                + [pltpu.SemaphoreType.DMA] * nbuf
            ),
        )

    parts = []
    for start, ipg in GROUPS:
        enc = pl.pallas_call(
            _tc_stage_body,
            grid=(ipg,),
            in_specs=[
                pl.BlockSpec((1, 3, 512, 512),
                             lambda n, s=start: (s + n, 0, 0, 0)),
            ] * 3,
            out_specs=pl.BlockSpec((1, 64, 4, 8, 128),
                                   lambda n: (n, 0, 0, 0, 0)),
            out_shape=jax.ShapeDtypeStruct((ipg, 64, 4, 8, 128), jnp.int32),
        )(pred, target, input_img)
        parts.append(sc_hists[ipg](enc.reshape(ipg * SPATIAL)))

    loss = pl.pallas_call(
        _finish_body,
        out_shape=jax.ShapeDtypeStruct((1, 1), f32),
    )(*parts)
    return loss[0, 0]


# speedup vs baseline: 1.0458x; 1.0458x over previous
import functools

import jax
import jax.numpy as jnp
from jax import lax
from jax.experimental import pallas as pl
from jax.experimental.pallas import tpu as pltpu
from jax.experimental.pallas import tpu_sc as plsc

NC = 2
NS = 16
L = 16
NW = NC * NS

NUM_IMGS = 16
SPATIAL = 512 * 512
G = 4
IPG = NUM_IMGS // G
GPIX = IPG * SPATIAL
PIX_PER_TILE = GPIX // NW
CH = 16384
NBUF = 2
STEPS = PIX_PER_TILE // CH
U = 8
TBL = 17 * L
TBL_P = 24 * L


def _tc_stage_body(p_ref, t_ref, x_ref, e_ref):
    p = p_ref[0]
    t = t_ref[0]
    x = x_ref[0]
    d = (0.299 * (p[0] - t[0]) + 0.587 * (p[1] - t[1])
         + 0.114 * (p[2] - t[2]))
    l16 = 4.784 * x[0] + 9.392 * x[1] + 1.824 * x[2]
    b = jnp.minimum(l16.astype(jnp.int32), 16)
    enc = (lax.bitcast_convert_type(d, jnp.int32) & ~31) | b
    for ct in range(4):
        sl = slice(128 * ct, 128 * (ct + 1))
        e_ref[0, :, ct] = enc[:, sl].reshape(64, 8, 128)


def _sc_body(e_hbm, out_ref, *scratch):
    wid = lax.axis_index("s") * NC + lax.axis_index("c")
    base = pl.multiple_of(wid * PIX_PER_TILE, 8)
    ebufs = scratch[:NBUF]
    accs = scratch[NBUF:NBUF + 4]
    stage = scratch[NBUF + 4]
    sems = scratch[NBUF + 5:]

    zero = jnp.zeros((L,), jnp.float32)
    for acc in accs:
        for j in range(TBL_P // L):
            acc[pl.ds(j * L, L)] = zero

    lane = lax.iota(jnp.int32, L)
    ones = jnp.ones((L,), jnp.float32)

    def start_dmas(step, slot):
        off = pl.multiple_of(base + step * CH, 8)
        pltpu.make_async_copy(e_hbm.at[pl.ds(off, CH)], ebufs[slot],
                              sems[slot]).start()

    def wait_dmas(slot):
        pltpu.make_async_copy(e_hbm.at[pl.ds(0, CH)], ebufs[slot],
                              sems[slot]).wait()

    def compute(slot):
        ebuf = ebufs[slot]

        @plsc.parallel_loop(0, CH // L, step=2, unroll=U // 2)
        def px(j):
            for u in range(2):
                ev = ebuf[pl.ds((j + u) * L, L)]
                idx = (ev & 31) * L + lane
                dv = plsc.bitcast(ev & ~31, jnp.float32)
                plsc.addupdate_scatter(accs[2 * u], [idx], ones)
                plsc.addupdate_scatter(accs[2 * u + 1], [idx], dv)

    for s in range(NBUF - 1):
        start_dmas(s, s)

    def step_body(t, _):
        step0 = t * NBUF
        for slot in range(NBUF):
            step = step0 + slot
            nxt = step + NBUF - 1

            @pl.when(nxt < STEPS)
            def _start_next():
                start_dmas(nxt, (NBUF - 1 + slot) % NBUF)

            wait_dmas(slot)
            compute(slot)
        return _

    lax.fori_loop(0, STEPS // NBUF, step_body, None)

    for q, acc in enumerate(accs):
        for j in range(TBL_P // L):
            stage[q * 3 + j // 8, pl.ds((j % 8) * L, L)] = acc[pl.ds(j * L, L)]
    pltpu.sync_copy(stage, out_ref.at[pl.ds(wid * 16, 16)])


def _finish_body(*refs):
    o_ref = refs[-1]
    x = refs[0][...]
    for r in refs[1:-1]:
        x = x + r[...]
    s = jnp.sum(x.reshape(NW, 16, 128), axis=0)
    cnt = s[0:3] + s[6:9]
    dd = s[3:6] + s[9:12]
    loss = jnp.zeros((1, 1), jnp.float32)
    for b in range(16):
        r = b // 8
        c0 = (b % 8) * L
        cb = jnp.sum(cnt[r:r + 1, c0:c0 + L], axis=1, keepdims=True)
        db = jnp.sum(dd[r:r + 1, c0:c0 + L], axis=1, keepdims=True)
        loss = loss + jnp.where(cb > 0.0,
                                jnp.abs(db) / jnp.maximum(cb, 1.0),
                                jnp.zeros_like(cb))
    o_ref[...] = loss / 16.0


def kernel(pred, target, input_img):
    f32 = jnp.float32
    mesh = plsc.VectorSubcoreMesh(core_axis_name="c", subcore_axis_name="s",
                                  num_cores=NC, num_subcores=NS)
    sc_hist = pl.kernel(
        _sc_body,
        out_type=jax.ShapeDtypeStruct((NW * 16, 128), f32),
        mesh=mesh,
        compiler_params=pltpu.CompilerParams(needs_layout_passes=False),
        scratch_types=(
            [pltpu.VMEM((CH,), jnp.int32)] * NBUF
            + [pltpu.VMEM((TBL_P,), f32)] * 4
            + [pltpu.VMEM((16, 128), f32)]
            + [pltpu.SemaphoreType.DMA] * NBUF
        ),
    )

    parts = []
    for g in range(G):
        enc = pl.pallas_call(
            _tc_stage_body,
            grid=(IPG,),
            in_specs=[
                pl.BlockSpec((1, 3, 512, 512),
                             lambda n, g=g: (g * IPG + n, 0, 0, 0)),
            ] * 3,
            out_specs=pl.BlockSpec((1, 64, 4, 8, 128),
                                   lambda n: (n, 0, 0, 0, 0)),
            out_shape=jax.ShapeDtypeStruct((IPG, 64, 4, 8, 128), jnp.int32),
        )(pred, target, input_img)
        parts.append(sc_hist(enc.reshape(GPIX)))

    loss = pl.pallas_call(
        _finish_body,
        out_shape=jax.ShapeDtypeStruct((1, 1), f32),
    )(*parts)
    return loss[0, 0]
